# Initial kernel scaffold; baseline (speedup 1.0000x reference)
import jax
import jax.numpy as jnp
from jax.experimental import pallas as pl


def kernel(inputs, edge_index, W_self0, W_neigh0, b0, W_self1, W_neigh1, b1, W_self2, W_neigh2, b2):
    raise NotImplementedError("write your pallas kernel here")


# speedup vs baseline: 4.2891x; 4.2891x over previous
import functools

import jax
import jax.numpy as jnp
from jax import lax
from jax.experimental import pallas as pl
from jax.experimental.pallas import tpu as pltpu
from jax.experimental.pallas import tpu_sc as plsc

N = 10000
E = 320000
D_IN = 128
D_H = 256
D_OUT = 64

NC = 2
NS = 16
NW = NC * NS

B = 128
ITERS = -(-E // (NW * B))
E_PAD = NW * ITERS * B
RPT = 632
N_PAD = NS * RPT
DUMMY = N

C0 = 144


def _sc_agg_body(K, *args):
    xs = args[:K]
    edges, zeros, out, e_v, rows_v, aggS, sem = args[K:]
    c = lax.axis_index("c")
    s = lax.axis_index("s")
    wid = c * NS + s
    r0 = s * RPT
    for k in range(K):
        pltpu.sync_copy(zeros.at[pl.ds(r0, RPT)], aggS.at[pl.ds(r0, RPT)])
        plsc.subcore_barrier()

        def it(i, carry, xk=xs[k]):
            pltpu.sync_copy(edges.at[wid, i], e_v)
            pltpu.async_copy(xk.at[e_v.at[0]], rows_v, sem).wait()
            pltpu.sync_copy(rows_v, aggS.at[e_v.at[1]], add=True)
            return carry

        lax.fori_loop(0, ITERS, it, 0)
        plsc.subcore_barrier()
        pltpu.sync_copy(aggS.at[pl.ds(r0, RPT)], out.at[c, k, pl.ds(r0, RPT)])
        if k + 1 < K:
            plsc.subcore_barrier()


@functools.cache
def _make_sc_agg(K, C):
    mesh = plsc.VectorSubcoreMesh(core_axis_name="c", subcore_axis_name="s")
    return pl.kernel(
        functools.partial(_sc_agg_body, K),
        out_type=jax.ShapeDtypeStruct((NC, K, N_PAD, C), jnp.float32),
        mesh=mesh,
        scratch_types=[
            pltpu.VMEM((2, B), jnp.int32),
            pltpu.VMEM((B, C), jnp.float32),
            pltpu.VMEM_SHARED((N_PAD, C), jnp.float32),
            pltpu.SemaphoreType.DMA,
        ],
        compiler_params=pltpu.CompilerParams(use_tc_tiling_on_sc=False),
    )


def _sc_agg(xchunks, edges):
    K = len(xchunks)
    C = xchunks[0].shape[1]
    zeros = jnp.zeros((N_PAD, C), jnp.float32)
    return _make_sc_agg(K, C)(*xchunks, edges, zeros)


def _deg_of(p0):
    return jnp.maximum(p0[0][:, 128:129] + p0[1][:, 128:129], 1.0)


def _mm0_body(x_ref, p_ref, ws_ref, wn_ref, b_ref, h_ref):
    p = p_ref[0] + p_ref[1]
    hn = p[:, :128] / _deg_of(p_ref)
    h = (jnp.dot(x_ref[...], ws_ref[...], preferred_element_type=jnp.float32)
         + jnp.dot(hn, wn_ref[...], preferred_element_type=jnp.float32)
         + b_ref[...])
    h_ref[...] = jnp.maximum(h, 0.0)


def _mm1_body(h0_ref, p1_ref, p0_ref, ws_ref, wn_ref, b_ref, wn2_ref,
              h1_ref, y2_ref):
    deg = _deg_of(p0_ref)
    hn = jnp.concatenate(
        [p1_ref[0, 0] + p1_ref[1, 0], p1_ref[0, 1] + p1_ref[1, 1]], axis=1) / deg
    h1 = (jnp.dot(h0_ref[...], ws_ref[...], preferred_element_type=jnp.float32)
          + jnp.dot(hn, wn_ref[...], preferred_element_type=jnp.float32)
          + b_ref[...])
    h1 = jnp.maximum(h1, 0.0)
    h1_ref[...] = h1
    y2_ref[...] = jnp.dot(h1, wn2_ref[...], preferred_element_type=jnp.float32)


def _mm2_body(h1_ref, p2_ref, p0_ref, ws_ref, b_ref, o_ref):
    hn = (p2_ref[0] + p2_ref[1]) / _deg_of(p0_ref)
    o_ref[...] = (jnp.dot(h1_ref[...], ws_ref[...],
                          preferred_element_type=jnp.float32)
                  + hn + b_ref[...])


BN = 1000
_G = N // BN


def _full(shape):
    return pl.BlockSpec(shape, lambda i: tuple(0 for _ in shape))


def _rows(shape):
    nd = len(shape)
    return pl.BlockSpec(shape, lambda i, nd=nd: tuple(
        i if d == nd - 2 else 0 for d in range(nd)))


def kernel(inputs, edge_index, W_self0, W_neigh0, b0, W_self1, W_neigh1, b1,
           W_self2, W_neigh2, b2):
    x = inputs
    pad = E_PAD - E
    src = jnp.concatenate([edge_index[0], jnp.zeros((pad,), jnp.int32)])
    dst = jnp.concatenate([edge_index[1], jnp.full((pad,), DUMMY, jnp.int32)])
    edges = (jnp.stack([src, dst])
             .reshape(2, NW, ITERS, B).transpose(1, 2, 0, 3))

    x_aug = jnp.concatenate(
        [x, jnp.ones((N, 1), jnp.float32), jnp.zeros((N, C0 - D_IN - 1), jnp.float32)],
        axis=1)
    p0 = _sc_agg([x_aug], edges)[:, 0]

    h0 = pl.pallas_call(
        _mm0_body,
        grid=(_G,),
        in_specs=[
            _rows((BN, D_IN)),
            _rows((2, BN, C0)),
            _full((D_IN, D_H)),
            _full((D_IN, D_H)),
            _full((1, D_H)),
        ],
        out_specs=_rows((BN, D_H)),
        out_shape=jax.ShapeDtypeStruct((N, D_H), jnp.float32),
    )(x, p0, W_self0, W_neigh0, b0.reshape(1, -1))

    h0a = lax.slice(h0, (0, 0), (N, 128))
    h0b = lax.slice(h0, (0, 128), (N, 256))
    p1 = _sc_agg([h0a, h0b], edges)

    h1, y2 = pl.pallas_call(
        _mm1_body,
        grid=(_G,),
        in_specs=[
            _rows((BN, D_H)),
            _rows((2, 2, BN, 128)),
            _rows((2, BN, C0)),
            _full((D_H, D_H)),
            _full((D_H, D_H)),
            _full((1, D_H)),
            _full((D_H, D_OUT)),
        ],
        out_specs=[_rows((BN, D_H)), _rows((BN, D_OUT))],
        out_shape=[jax.ShapeDtypeStruct((N, D_H), jnp.float32),
                   jax.ShapeDtypeStruct((N, D_OUT), jnp.float32)],
    )(h0, p1, p0, W_self1, W_neigh1, b1.reshape(1, -1), W_neigh2)

    p2 = _sc_agg([y2], edges)[:, 0]

    out = pl.pallas_call(
        _mm2_body,
        grid=(_G,),
        in_specs=[
            _rows((BN, D_H)),
            _rows((2, BN, D_OUT)),
            _rows((2, BN, C0)),
            _full((D_H, D_OUT)),
            _full((1, D_OUT)),
        ],
        out_specs=_rows((BN, D_OUT)),
        out_shape=jax.ShapeDtypeStruct((N, D_OUT), jnp.float32),
    )(h1, p2, p0, W_self2, b2.reshape(1, -1))

    return (out, h0, h1)
